# Initial kernel scaffold; baseline (speedup 1.0000x reference)
import jax
import jax.numpy as jnp
from jax.experimental import pallas as pl


def kernel(x, table):
    raise NotImplementedError("write your pallas kernel here")


# speedup vs baseline: 3.9927x; 3.9927x over previous
import functools

import numpy as np
import jax
import jax.numpy as jnp
from jax import lax
from jax.experimental import pallas as pl
from jax.experimental.pallas import tpu as pltpu
from jax.experimental.pallas import tpu_sc as plsc

_VOCAB = 100000
_SEQ = 200
_D = 64
_B = 4096
_NC, _NS = 2, 16
_NW = _NC * _NS
_BPW = _B // _NW
_HALF = _SEQ // 2


def _pos_enc():
    pos = np.arange(_SEQ, dtype=np.float32)[:, None]
    i = np.arange(_D, dtype=np.float32)[None, :]
    rates = 1.0 / np.power(10000.0, (2.0 * np.floor(i / 2.0)) / _D)
    ang = pos * rates
    pe = np.zeros((_SEQ, _D), np.float32)
    pe[:, 0::2] = np.sin(ang[:, 0::2])
    pe[:, 1::2] = np.cos(ang[:, 1::2])
    return pe


def _body(x2, pe_h, tab, out, idx_all, pe_v, rows0, rows1, g0, g1, s0, s1):
    wid = lax.axis_index("s") * _NC + lax.axis_index("c")
    pltpu.sync_copy(x2.at[pl.ds(wid * 2 * _BPW, 2 * _BPW)], idx_all)
    pltpu.sync_copy(pe_h, pe_v)
    rows = (rows0, rows1)
    gs = (g0, g1)
    ss = (s0, s1)
    out_base = wid * _BPW

    def gather_start(r, b):
        pltpu.make_async_copy(
            tab.at[idx_all.at[2 * r]], rows[b].at[pl.ds(0, _HALF)], gs[b]
        ).start()
        pltpu.make_async_copy(
            tab.at[idx_all.at[2 * r + 1]], rows[b].at[pl.ds(_HALF, _HALF)], gs[b]
        ).start()

    def gather_wait(b):
        pltpu.make_async_copy(
            tab.at[idx_all.at[0]], rows[b].at[pl.ds(0, _HALF)], gs[b]
        ).wait()
        pltpu.make_async_copy(
            tab.at[idx_all.at[1]], rows[b].at[pl.ds(_HALF, _HALF)], gs[b]
        ).wait()

    def store_start(r, b):
        pltpu.make_async_copy(
            rows[b], out.at[pl.ds((out_base + r) * _SEQ, _SEQ)], ss[b]
        ).start()

    def store_wait(b):
        pltpu.make_async_copy(rows[b], out.at[pl.ds(0, _SEQ)], ss[b]).wait()

    gather_start(0, 0)

    def step(r2, carry):
        for b in range(2):
            r = r2 + b
            nb = 1 - b

            @pl.when(r + 1 < _BPW)
            def _():
                @pl.when(r >= 1)
                def _():
                    store_wait(nb)

                gather_start(r + 1, nb)

            gather_wait(b)

            def add_row(rr, c2):
                for c in range(_D // 16):
                    plsc.addupdate(
                        rows[b].at[rr, pl.ds(c * 16, 16)],
                        pe_v[rr, pl.ds(c * 16, 16)],
                    )
                return c2

            lax.fori_loop(0, _SEQ, add_row, 0)
            store_start(r, b)
        return carry

    lax.fori_loop(0, _BPW // 2, lambda i, c: step(i * 2, c), 0)
    store_wait(0)
    store_wait(1)


@jax.jit
def kernel(x, table):
    x2 = x.astype(jnp.int32).reshape(_B * 2, _SEQ // 2)
    pe = jnp.asarray(_pos_enc())
    mesh = plsc.VectorSubcoreMesh(
        core_axis_name="c", subcore_axis_name="s", num_cores=_NC, num_subcores=_NS
    )
    run = pl.kernel(
        _body,
        out_type=jax.ShapeDtypeStruct((_B * _SEQ, _D), jnp.float32),
        mesh=mesh,
        compiler_params=pltpu.CompilerParams(use_tc_tiling_on_sc=False),
        scratch_types=[
            pltpu.VMEM((2 * _BPW, _HALF), jnp.int32),
            pltpu.VMEM((_SEQ, _D), jnp.float32),
            pltpu.VMEM((_SEQ, _D), jnp.float32),
            pltpu.VMEM((_SEQ, _D), jnp.float32),
            pltpu.SemaphoreType.DMA,
            pltpu.SemaphoreType.DMA,
            pltpu.SemaphoreType.DMA,
            pltpu.SemaphoreType.DMA,
        ],
    )
    out = run(x2, pe, table)
    return out.reshape(_B, _SEQ, _D)
